# baseline (speedup 1.0000x reference)
import functools

import jax
import jax.numpy as jnp
from jax import lax
from jax.experimental import pallas as pl
from jax.experimental.pallas import tpu as pltpu
from jax.experimental.pallas import tpu_sc as plsc

N_DRUG = 25000
N_DIS = 25000
N_NODE = 25000
N_EDGE = 300000
D_IN = 256
D = 128
H = 4
DH = 32
L = 2
B = 16384

NC = 2
NS = 16
NP = 311296
CHUNK = NP // NS
SBN = 512
SB = CHUNK // SBN
W = 32
WPB = SBN // W
HALF = N_NODE // 2
ACCR = 12544
ACCW = 144
DENW = 16
DENR = 3200
DROWS_PER_TILE = DENR // NS
ROWS_PER_TILE = ACCR // NS

_SC_MESH = plsc.VectorSubcoreMesh(core_axis_name="c", subcore_axis_name="s")
_STAGE = -2


def _gelu_k(x):
    return x * 0.5 * (1.0 + lax.erf(x * 0.7071067811865476))


def _projkqv_body(x_ref, win_ref, bin_ref, wk_ref, bk_ref, wq_ref, bq_ref,
                  wv_ref, bv_ref, xs_ref, k_ref, q_ref, v_ref):
    xs = jnp.dot(x_ref[...], win_ref[...], preferred_element_type=jnp.float32) + bin_ref[...]
    xs_ref[...] = xs
    k_ref[...] = jnp.dot(xs, wk_ref[...], preferred_element_type=jnp.float32) + bk_ref[...]
    q_ref[...] = jnp.dot(xs, wq_ref[...], preferred_element_type=jnp.float32) + bq_ref[...]
    v_ref[...] = jnp.dot(xs, wv_ref[...], preferred_element_type=jnp.float32) + bv_ref[...]


def _proj_kqv(x, w_in, b_in, wk, bk, wq, bq, wv, bv):
    blk = 1000
    grid = (N_NODE // blk,)
    wspec = pl.BlockSpec((D, D), lambda i: (0, 0))
    bspec = pl.BlockSpec((D,), lambda i: (0,))
    outs = pl.pallas_call(
        _projkqv_body,
        grid=grid,
        in_specs=[
            pl.BlockSpec((blk, D_IN), lambda i: (i, 0)),
            pl.BlockSpec((D_IN, D), lambda i: (0, 0)),
            bspec, wspec, bspec, wspec, bspec, wspec, bspec,
        ],
        out_specs=[pl.BlockSpec((blk, D), lambda i: (i, 0))] * 4,
        out_shape=[jax.ShapeDtypeStruct((N_NODE, D), jnp.float32)] * 4,
    )(x, w_in, b_in, wk, bk, wq, bq, wv, bv)
    return outs


def _kqv_body(xs_ref, wk_ref, bk_ref, wq_ref, bq_ref, wv_ref, bv_ref,
              k_ref, q_ref, v_ref):
    xs = xs_ref[...]
    k_ref[...] = jnp.dot(xs, wk_ref[...], preferred_element_type=jnp.float32) + bk_ref[...]
    q_ref[...] = jnp.dot(xs, wq_ref[...], preferred_element_type=jnp.float32) + bq_ref[...]
    v_ref[...] = jnp.dot(xs, wv_ref[...], preferred_element_type=jnp.float32) + bv_ref[...]


def _kqv(xs, wk, bk, wq, bq, wv, bv):
    blk = 1000
    grid = (N_NODE // blk,)
    wspec = pl.BlockSpec((D, D), lambda i: (0, 0))
    bspec = pl.BlockSpec((D,), lambda i: (0,))
    return pl.pallas_call(
        _kqv_body,
        grid=grid,
        in_specs=[pl.BlockSpec((blk, D), lambda i: (i, 0)),
                  wspec, bspec, wspec, bspec, wspec, bspec],
        out_specs=[pl.BlockSpec((blk, D), lambda i: (i, 0))] * 3,
        out_shape=[jax.ShapeDtypeStruct((N_NODE, D), jnp.float32)] * 3,
    )(xs, wk, bk, wq, bq, wv, bv)


def _post_body(acc_ref, den_ref, xs_ref, wo_ref, bo_ref, beta_ref, lng_ref,
               lnb_ref, out_ref):
    blk = acc_ref.shape[0]
    agg = acc_ref[...]
    den = den_ref[...] + 1e-16
    drep = jnp.broadcast_to(den[:, :, None], (blk, H, DH)).reshape(blk, D)
    g = _gelu_k(agg / drep)
    o = jnp.dot(g, wo_ref[...], preferred_element_type=jnp.float32) + bo_ref[...]
    beta = beta_ref[0, 0]
    u = beta * o + (2.0 - beta) * xs_ref[...]
    mu = jnp.mean(u, axis=-1, keepdims=True)
    var = jnp.mean((u - mu) ** 2, axis=-1, keepdims=True)
    ln = (u - mu) / jnp.sqrt(var + 1e-5) * lng_ref[...] + lnb_ref[...]
    out_ref[...] = _gelu_k(ln)


def _post(acc, den, xs, wo, bo, beta, lng, lnb):
    blk = 1000
    grid = (N_NODE // blk,)
    out = pl.pallas_call(
        _post_body,
        grid=grid,
        in_specs=[
            pl.BlockSpec((blk, D), lambda i: (i, 0)),
            pl.BlockSpec((blk, H), lambda i: (i, 0)),
            pl.BlockSpec((blk, D), lambda i: (i, 0)),
            pl.BlockSpec((D, D), lambda i: (0, 0)),
            pl.BlockSpec((D,), lambda i: (0,)),
            pl.BlockSpec(memory_space=pltpu.SMEM),
            pl.BlockSpec((D,), lambda i: (0,)),
            pl.BlockSpec((D,), lambda i: (0,)),
        ],
        out_specs=pl.BlockSpec((blk, D), lambda i: (i, 0)),
        out_shape=jax.ShapeDtypeStruct((N_NODE, D), jnp.float32),
    )(acc, den, xs, wo, bo, beta, lng, lnb)
    return out


def _mlp_body(g0_ref, g1_ref, w1_ref, b1_ref, w2_ref, b2_ref, w3_ref, b3_ref,
              out_ref):
    x = g0_ref[...] * g1_ref[...]
    h = _gelu_k(jnp.dot(x, w1_ref[...], preferred_element_type=jnp.float32) + b1_ref[...])
    h = _gelu_k(jnp.dot(h, w2_ref[...], preferred_element_type=jnp.float32) + b2_ref[...])
    out_ref[...] = jnp.dot(h, w3_ref[...], preferred_element_type=jnp.float32) + b3_ref[...]


def _decoder_mlp(g0, g1, W1, b1, W2, b2, W3, b3):
    blk = 2048
    grid = (B // blk,)
    out = pl.pallas_call(
        _mlp_body,
        grid=grid,
        in_specs=[
            pl.BlockSpec((blk, D), lambda i: (i, 0)),
            pl.BlockSpec((blk, D), lambda i: (i, 0)),
            pl.BlockSpec((D, 256), lambda i: (0, 0)),
            pl.BlockSpec((256,), lambda i: (0,)),
            pl.BlockSpec((256, 128), lambda i: (0, 0)),
            pl.BlockSpec((128,), lambda i: (0,)),
            pl.BlockSpec((128, 1), lambda i: (0, 0)),
            pl.BlockSpec((1,), lambda i: (0,)),
        ],
        out_specs=pl.BlockSpec((blk, 1), lambda i: (i, 0)),
        out_shape=jax.ShapeDtypeStruct((B, 1), jnp.float32),
    )(g0, g1, W1, b1, W2, b2, W3, b3)
    return out[:, 0]


def _perm(v, idx):
    return lax.gather(
        v, idx[:, None],
        dimension_numbers=lax.GatherDimensionNumbers(
            offset_dims=(), collapsed_slice_dims=(0,), start_index_map=(0,)),
        slice_sizes=(1,), mode=lax.GatherScatterMode.PROMISE_IN_BOUNDS)


def _hsum_splat(v, lane):
    for k in (1, 2, 4, 8):
        v = v + _perm(v, lane ^ k)
    return v


def _edges_body(x_ref, out_ref):
    out_ref[...] = x_ref[...].reshape(out_ref.shape)


def _relayout_edges(x):
    rows = NP // SBN
    blk = 16
    return pl.pallas_call(
        _edges_body,
        grid=(rows // blk,),
        in_specs=[pl.BlockSpec((blk * SBN,), lambda i: (i,))],
        out_specs=pl.BlockSpec((blk, SBN), lambda i: (i, 0)),
        out_shape=jax.ShapeDtypeStruct((rows, SBN), jnp.int32),
    )(x)


@functools.partial(
    pl.kernel,
    out_type=[jax.ShapeDtypeStruct((NC, ACCR, D), jnp.float32),
              jax.ShapeDtypeStruct((NC, DENR, DENW), jnp.float32)],
    mesh=_SC_MESH,
    scratch_types=[
        pltpu.VMEM((SBN,), jnp.int32),
        pltpu.VMEM((SBN,), jnp.int32),
        pltpu.VMEM((W, D), jnp.float32),
        pltpu.VMEM((W, D), jnp.float32),
        pltpu.VMEM((W, D), jnp.float32),
        pltpu.VMEM((W, DENW), jnp.float32),
        pltpu.VMEM((8, D), jnp.float32),
        pltpu.VMEM((16, DENW), jnp.float32),
        pltpu.VMEM((8, DENW), jnp.float32),
        pltpu.VMEM((W,), jnp.int32),
        pltpu.VMEM((W,), jnp.int32),
        pltpu.VMEM((W,), jnp.int32),
        pltpu.VMEM((W,), jnp.int32),
        pltpu.VMEM_SHARED((ACCR, D), jnp.float32),
        pltpu.VMEM_SHARED((DENR, DENW), jnp.float32),
        pltpu.SemaphoreType.DMA,
        pltpu.SemaphoreType.DMA,
        pltpu.SemaphoreType.DMA,
    ],
)
def _edge_kernel(kp, qm, vp, srcm, dstm, out_agg, out_den,
                 src_st, dst_st, kbuf, qbuf, upd, dupd,
                 zbuf, dzbuf, dzbuf2, sidx, qiw, dstw, dstw2,
                 acc_sh, den_sh, sem_k, sem_q, sem_v):
    c = lax.axis_index("c")
    s = lax.axis_index("s")
    lo = c * HALF
    lane16 = lax.iota(jnp.int32, 16)
    lmod4 = lane16 & 3
    lgrp4 = lane16 >> 2

    zv = jnp.zeros((16,), jnp.float32)
    for r in range(8):
        for j in range(D // 16):
            zbuf[r, pl.ds(j * 16, 16)] = zv
        dzbuf[r, pl.ds(0, 16)] = zv
        dzbuf[r + 8, pl.ds(0, 16)] = zv
        dzbuf2[r, pl.ds(0, 16)] = zv
    r0 = s * ROWS_PER_TILE
    d0 = s * DROWS_PER_TILE
    if _STAGE >= 1:
        for t in range(ROWS_PER_TILE // 8):
            pltpu.sync_copy(zbuf, acc_sh.at[pl.ds(r0 + t * 8, 8)])
        for t in range(DROWS_PER_TILE // 16):
            pltpu.sync_copy(dzbuf, den_sh.at[pl.ds(d0 + t * 16, 16)])
        pltpu.sync_copy(dzbuf2,
                        den_sh.at[pl.ds(d0 + (DROWS_PER_TILE // 16) * 16, 8)])
    if _STAGE >= 2:
        plsc.subcore_barrier()

    def _block(sb, carry0):
        row = s * SB + sb
        pltpu.sync_copy(srcm.at[row], src_st)
        pltpu.sync_copy(dstm.at[row], dst_st)

        def _window(w, carry):
            base = pl.multiple_of(w * W, W)
            for i in range(W // 16):
                dv = dst_st[pl.ds(base + i * 16, 16)]
                sv = src_st[pl.ds(base + i * 16, 16)]
                m = (dv >= lo) & (dv < lo + HALF)
                dloc = jnp.where(m, dv - lo, jnp.int32(HALF))
                sidx[pl.ds(i * 16, 16)] = sv
                qiw[pl.ds(i * 16, 16)] = jnp.minimum(dv, jnp.int32(N_NODE - 1))
                dstw[pl.ds(i * 16, 16)] = dloc
                dstw2[pl.ds(i * 16, 16)] = dloc >> 2
            if _STAGE >= 5:
                cpk = pltpu.async_copy(kp.at[sidx], kbuf, sem_k)
                cpq = pltpu.async_copy(qm.at[qiw], qbuf, sem_q)
                cpv = pltpu.async_copy(vp.at[sidx], upd, sem_v)
                cpk.wait()
                cpq.wait()
                cpv.wait()

            if _STAGE >= 6:
                def _group(g, carry2):
                    goff = pl.multiple_of(g * 16, 16)
                    dvec = dstw[pl.ds(goff, 16)]
                    for j in range(16):
                        e = goff + j
                        ex = []
                        for h in range(H):
                            k0 = kbuf[e, pl.ds(h * 32, 16)]
                            k1 = kbuf[e, pl.ds(h * 32 + 16, 16)]
                            q0 = qbuf[e, pl.ds(h * 32, 16)]
                            q1 = qbuf[e, pl.ds(h * 32 + 16, 16)]
                            th = k0 * q0 + k1 * q1
                            ex.append(jnp.exp(_hsum_splat(th, lane16)))
                        for i in range(D // 16):
                            upd[e, pl.ds(i * 16, 16)] = upd[e, pl.ds(i * 16, 16)] * ex[i // 2]
                        exr = jnp.where(lmod4 == 0, ex[0],
                                        jnp.where(lmod4 == 1, ex[1],
                                                  jnp.where(lmod4 == 2, ex[2], ex[3])))
                        dsplat = _perm(dvec, jnp.full((16,), j, jnp.int32))
                        dmod = dsplat & 3
                        dupd[e, pl.ds(0, 16)] = jnp.where(lgrp4 == dmod, exr, 0.0)
                    return carry2

                lax.fori_loop(0, W // 16, _group, 0)
            if _STAGE >= 7:
                pltpu.sync_copy(upd, acc_sh.at[dstw], add=True)
            if _STAGE >= 8:
                pltpu.sync_copy(dupd, den_sh.at[dstw2], add=True)
            return carry

        if _STAGE >= 5:
            lax.fori_loop(0, WPB, _window, 0)
        return carry0

    if _STAGE >= 4:
        lax.fori_loop(0, SB, _block, 0)

    if _STAGE >= 2:
        plsc.subcore_barrier()
    if _STAGE >= 3:
        for cc in range(NC):
            @pl.when(c == cc)
            def _():
                pltpu.sync_copy(acc_sh.at[pl.ds(r0, ROWS_PER_TILE)],
                                out_agg.at[cc, pl.ds(r0, ROWS_PER_TILE)])
                pltpu.sync_copy(den_sh.at[pl.ds(d0, DROWS_PER_TILE)],
                                out_den.at[cc, pl.ds(d0, DROWS_PER_TILE)])


@functools.partial(
    pl.kernel,
    out_type=jax.ShapeDtypeStruct((B, D), jnp.float32),
    mesh=_SC_MESH,
    scratch_types=[
        pltpu.VMEM((B // (NC * NS),), jnp.int32),
        pltpu.VMEM((128,), jnp.int32),
        pltpu.VMEM((128, D), jnp.float32),
        pltpu.SemaphoreType.DMA,
    ],
)
def _gather_kernel(table, idx, out, ibuf, iwin, rbuf, sem):
    wid = lax.axis_index("s") * NC + lax.axis_index("c")
    per = B // (NC * NS)
    base = wid * per
    pltpu.sync_copy(idx.at[pl.ds(base, per)], ibuf)
    for w in range(per // 128):
        for i in range(8):
            iwin[pl.ds(i * 16, 16)] = ibuf[pl.ds(w * 128 + i * 16, 16)]
        pltpu.async_copy(table.at[iwin], rbuf, sem).wait()
        pltpu.sync_copy(rbuf, out.at[pl.ds(base + w * 128, 128)])


def _block_diag(rel):
    eye = jnp.eye(H, dtype=jnp.float32)
    t = eye[None, None, :, None, :, None] * rel[:, :, :, :, None, :]
    return t.reshape(L, 2, D, D)


def kernel(x_drug, x_disease, W_in, b_in, Wk, bk, Wq, bq, Wv, bv, Wo, bo,
           a_rel, m_rel, p_rel, skip, lng, lnb, W1, b1, W2, b2, W3, b3,
           ei_treats, ei_rev, drug_index, disease_index):
    f32 = jnp.float32
    Abd = _block_diag(a_rel)
    Mbd = _block_diag(m_rel)
    Wk_f = jnp.einsum('ltij,ltjk->ltik', Wk, Abd)
    bk_f = jnp.einsum('ltj,ltjk->ltk', bk, Abd)
    Wv_f = jnp.einsum('ltij,ltjk->ltik', Wv, Mbd)
    bv_f = jnp.einsum('ltj,ltjk->ltk', bv, Mbd)
    qs = jnp.repeat(p_rel[:, ::-1, :], DH, axis=-1) * (1.0 / jnp.sqrt(DH))
    Wq_f = Wq * qs[:, :, None, :]
    bq_f = bq * qs
    beta = jax.nn.sigmoid(skip)

    npad = NP - N_EDGE
    srcs, dsts = [], []
    for ei in (ei_treats, ei_rev):
        srcs.append(_relayout_edges(
            jnp.concatenate([ei[0], jnp.zeros((npad,), jnp.int32)])))
        dsts.append(_relayout_edges(
            jnp.concatenate([ei[1], jnp.full((npad,), N_NODE, jnp.int32)])))

    edge_idx = [ei_treats, ei_rev]
    xs = [None, None]
    K = [None, None]
    Q = [None, None]
    V = [None, None]
    x_in = [x_drug, x_disease]
    for l in range(L):
        for t in range(2):
            if l == 0:
                xs[t], K[t], Q[t], V[t] = _proj_kqv(
                    x_in[t], W_in[t], b_in[t],
                    Wk_f[l, t], bk_f[l, t], Wq_f[l, t], bq_f[l, t],
                    Wv_f[l, t], bv_f[l, t])
            else:
                K[t], Q[t], V[t] = _kqv(
                    xs[t], Wk_f[l, t], bk_f[l, t], Wq_f[l, t], bq_f[l, t],
                    Wv_f[l, t], bv_f[l, t])
        acc = [None, None]
        den = [None, None]
        for e in range(2):
            if _STAGE == -2:
                src, dst = edge_idx[e][0], edge_idx[e][1]
                k_e = K[e].reshape(N_NODE, H, DH)[src]
                v_e = V[e].reshape(N_NODE, H, DH)[src]
                q_e = Q[1 - e].reshape(N_NODE, H, DH)[dst]
                exv = jnp.exp(jnp.sum(q_e * k_e, axis=-1))
                den[e] = jax.ops.segment_sum(exv, dst, num_segments=N_NODE)
                acc[e] = jax.ops.segment_sum(
                    v_e * exv[:, :, None], dst, num_segments=N_NODE
                ).reshape(N_NODE, D)
            else:
                a2, d2 = _edge_kernel(K[e], Q[1 - e], V[e], srcs[e], dsts[e])
                acc[e] = jnp.concatenate([a2[0, :HALF], a2[1, :HALF]], axis=0)
                d2r = d2.reshape(NC, DENR * 4, H)
                den[e] = jnp.concatenate([d2r[0, :HALF], d2r[1, :HALF]], axis=0)
        new_xs = []
        for t in range(2):
            beta_t = beta[l, t].astype(f32).reshape(1, 1)
            new_xs.append(_post(acc[1 - t], den[1 - t], xs[t], Wo[l, t],
                                bo[l, t], beta_t, lng[l, t], lnb[l, t]))
        xs = new_xs

    g0 = _gather_kernel(xs[0], drug_index)
    g1 = _gather_kernel(xs[1], disease_index)
    return _decoder_mlp(g0, g1, W1, b1, W2, b2, W3, b3)


# speedup vs baseline: 1.0000x; 1.0000x over previous
import functools

import jax
import jax.numpy as jnp
from jax import lax
from jax.experimental import pallas as pl
from jax.experimental.pallas import tpu as pltpu
from jax.experimental.pallas import tpu_sc as plsc

N_DRUG = 25000
N_DIS = 25000
N_NODE = 25000
N_EDGE = 300000
D_IN = 256
D = 128
H = 4
DH = 32
L = 2
B = 16384

NC = 2
NS = 16
NP = 311296
CHUNK = NP // NS
SBN = 512
SB = CHUNK // SBN
W = 32
WPB = SBN // W
HALF = N_NODE // 2
ACCR = 12544
ACCW = 144
DENW = 16
DENR = 3200
DROWS_PER_TILE = DENR // NS
ROWS_PER_TILE = ACCR // NS

_SC_MESH = plsc.VectorSubcoreMesh(core_axis_name="c", subcore_axis_name="s")
_STAGE = 8


def _gelu_k(x):
    return x * 0.5 * (1.0 + lax.erf(x * 0.7071067811865476))


def _projkqv_body(x_ref, win_ref, bin_ref, wk_ref, bk_ref, wq_ref, bq_ref,
                  wv_ref, bv_ref, xs_ref, k_ref, q_ref, v_ref):
    xs = jnp.dot(x_ref[...], win_ref[...], preferred_element_type=jnp.float32) + bin_ref[...]
    xs_ref[...] = xs
    k_ref[...] = jnp.dot(xs, wk_ref[...], preferred_element_type=jnp.float32) + bk_ref[...]
    q_ref[...] = jnp.dot(xs, wq_ref[...], preferred_element_type=jnp.float32) + bq_ref[...]
    v_ref[...] = jnp.dot(xs, wv_ref[...], preferred_element_type=jnp.float32) + bv_ref[...]


def _proj_kqv(x, w_in, b_in, wk, bk, wq, bq, wv, bv):
    blk = 1000
    grid = (N_NODE // blk,)
    wspec = pl.BlockSpec((D, D), lambda i: (0, 0))
    bspec = pl.BlockSpec((D,), lambda i: (0,))
    outs = pl.pallas_call(
        _projkqv_body,
        grid=grid,
        in_specs=[
            pl.BlockSpec((blk, D_IN), lambda i: (i, 0)),
            pl.BlockSpec((D_IN, D), lambda i: (0, 0)),
            bspec, wspec, bspec, wspec, bspec, wspec, bspec,
        ],
        out_specs=[pl.BlockSpec((blk, D), lambda i: (i, 0))] * 4,
        out_shape=[jax.ShapeDtypeStruct((N_NODE, D), jnp.float32)] * 4,
    )(x, w_in, b_in, wk, bk, wq, bq, wv, bv)
    return outs


def _kqv_body(xs_ref, wk_ref, bk_ref, wq_ref, bq_ref, wv_ref, bv_ref,
              k_ref, q_ref, v_ref):
    xs = xs_ref[...]
    k_ref[...] = jnp.dot(xs, wk_ref[...], preferred_element_type=jnp.float32) + bk_ref[...]
    q_ref[...] = jnp.dot(xs, wq_ref[...], preferred_element_type=jnp.float32) + bq_ref[...]
    v_ref[...] = jnp.dot(xs, wv_ref[...], preferred_element_type=jnp.float32) + bv_ref[...]


def _kqv(xs, wk, bk, wq, bq, wv, bv):
    blk = 1000
    grid = (N_NODE // blk,)
    wspec = pl.BlockSpec((D, D), lambda i: (0, 0))
    bspec = pl.BlockSpec((D,), lambda i: (0,))
    return pl.pallas_call(
        _kqv_body,
        grid=grid,
        in_specs=[pl.BlockSpec((blk, D), lambda i: (i, 0)),
                  wspec, bspec, wspec, bspec, wspec, bspec],
        out_specs=[pl.BlockSpec((blk, D), lambda i: (i, 0))] * 3,
        out_shape=[jax.ShapeDtypeStruct((N_NODE, D), jnp.float32)] * 3,
    )(xs, wk, bk, wq, bq, wv, bv)


def _post_body(acc_ref, den_ref, xs_ref, wo_ref, bo_ref, beta_ref, lng_ref,
               lnb_ref, out_ref):
    blk = acc_ref.shape[0]
    agg = acc_ref[...]
    den = den_ref[...] + 1e-16
    drep = jnp.broadcast_to(den[:, :, None], (blk, H, DH)).reshape(blk, D)
    g = _gelu_k(agg / drep)
    o = jnp.dot(g, wo_ref[...], preferred_element_type=jnp.float32) + bo_ref[...]
    beta = beta_ref[0, 0]
    u = beta * o + (2.0 - beta) * xs_ref[...]
    mu = jnp.mean(u, axis=-1, keepdims=True)
    var = jnp.mean((u - mu) ** 2, axis=-1, keepdims=True)
    ln = (u - mu) / jnp.sqrt(var + 1e-5) * lng_ref[...] + lnb_ref[...]
    out_ref[...] = _gelu_k(ln)


def _post(acc, den, xs, wo, bo, beta, lng, lnb):
    blk = 1000
    grid = (N_NODE // blk,)
    out = pl.pallas_call(
        _post_body,
        grid=grid,
        in_specs=[
            pl.BlockSpec((blk, D), lambda i: (i, 0)),
            pl.BlockSpec((blk, H), lambda i: (i, 0)),
            pl.BlockSpec((blk, D), lambda i: (i, 0)),
            pl.BlockSpec((D, D), lambda i: (0, 0)),
            pl.BlockSpec((D,), lambda i: (0,)),
            pl.BlockSpec(memory_space=pltpu.SMEM),
            pl.BlockSpec((D,), lambda i: (0,)),
            pl.BlockSpec((D,), lambda i: (0,)),
        ],
        out_specs=pl.BlockSpec((blk, D), lambda i: (i, 0)),
        out_shape=jax.ShapeDtypeStruct((N_NODE, D), jnp.float32),
    )(acc, den, xs, wo, bo, beta, lng, lnb)
    return out


def _mlp_body(g0_ref, g1_ref, w1_ref, b1_ref, w2_ref, b2_ref, w3_ref, b3_ref,
              out_ref):
    x = g0_ref[...] * g1_ref[...]
    h = _gelu_k(jnp.dot(x, w1_ref[...], preferred_element_type=jnp.float32) + b1_ref[...])
    h = _gelu_k(jnp.dot(h, w2_ref[...], preferred_element_type=jnp.float32) + b2_ref[...])
    out_ref[...] = jnp.dot(h, w3_ref[...], preferred_element_type=jnp.float32) + b3_ref[...]


def _decoder_mlp(g0, g1, W1, b1, W2, b2, W3, b3):
    blk = 2048
    grid = (B // blk,)
    out = pl.pallas_call(
        _mlp_body,
        grid=grid,
        in_specs=[
            pl.BlockSpec((blk, D), lambda i: (i, 0)),
            pl.BlockSpec((blk, D), lambda i: (i, 0)),
            pl.BlockSpec((D, 256), lambda i: (0, 0)),
            pl.BlockSpec((256,), lambda i: (0,)),
            pl.BlockSpec((256, 128), lambda i: (0, 0)),
            pl.BlockSpec((128,), lambda i: (0,)),
            pl.BlockSpec((128, 1), lambda i: (0, 0)),
            pl.BlockSpec((1,), lambda i: (0,)),
        ],
        out_specs=pl.BlockSpec((blk, 1), lambda i: (i, 0)),
        out_shape=jax.ShapeDtypeStruct((B, 1), jnp.float32),
    )(g0, g1, W1, b1, W2, b2, W3, b3)
    return out[:, 0]


def _perm(v, idx):
    return lax.gather(
        v, idx[:, None],
        dimension_numbers=lax.GatherDimensionNumbers(
            offset_dims=(), collapsed_slice_dims=(0,), start_index_map=(0,)),
        slice_sizes=(1,), mode=lax.GatherScatterMode.PROMISE_IN_BOUNDS)


def _hsum_splat(v, lane):
    for k in (1, 2, 4, 8):
        v = v + _perm(v, lane ^ k)
    return v


def _edges_body(x_ref, out_ref):
    out_ref[...] = x_ref[...].reshape(out_ref.shape)


def _relayout_edges(x):
    rows = NP // SBN
    blk = 16
    return pl.pallas_call(
        _edges_body,
        grid=(rows // blk,),
        in_specs=[pl.BlockSpec((blk * SBN,), lambda i: (i,))],
        out_specs=pl.BlockSpec((blk, SBN), lambda i: (i, 0)),
        out_shape=jax.ShapeDtypeStruct((rows, SBN), jnp.int32),
    )(x)


@functools.partial(
    pl.kernel,
    out_type=[jax.ShapeDtypeStruct((NC, ACCR, D), jnp.float32),
              jax.ShapeDtypeStruct((NC, DENR, DENW), jnp.float32)],
    mesh=_SC_MESH,
    scratch_types=[
        pltpu.VMEM((SBN,), jnp.int32),
        pltpu.VMEM((SBN,), jnp.int32),
        pltpu.VMEM((W, D), jnp.float32),
        pltpu.VMEM((W, D), jnp.float32),
        pltpu.VMEM((W, D), jnp.float32),
        pltpu.VMEM((W, DENW), jnp.float32),
        pltpu.VMEM((8, D), jnp.float32),
        pltpu.VMEM((16, DENW), jnp.float32),
        pltpu.VMEM((8, DENW), jnp.float32),
        pltpu.VMEM((W,), jnp.int32),
        pltpu.VMEM((W,), jnp.int32),
        pltpu.VMEM((W,), jnp.int32),
        pltpu.VMEM((W,), jnp.int32),
        pltpu.VMEM_SHARED((ACCR, D), jnp.float32),
        pltpu.VMEM_SHARED((DENR, DENW), jnp.float32),
        pltpu.SemaphoreType.DMA,
        pltpu.SemaphoreType.DMA,
        pltpu.SemaphoreType.DMA,
    ],
)
def _edge_kernel(kp, qm, vp, srcm, dstm, out_agg, out_den,
                 src_st, dst_st, kbuf, qbuf, upd, dupd,
                 zbuf, dzbuf, dzbuf2, sidx, qiw, dstw, dstw2,
                 acc_sh, den_sh, sem_k, sem_q, sem_v):
    c = lax.axis_index("c")
    s = lax.axis_index("s")
    lo = c * HALF
    lane16 = lax.iota(jnp.int32, 16)
    lmod4 = lane16 & 3
    lgrp4 = lane16 >> 2

    zv = jnp.zeros((16,), jnp.float32)
    for r in range(8):
        for j in range(D // 16):
            zbuf[r, pl.ds(j * 16, 16)] = zv
        dzbuf[r, pl.ds(0, 16)] = zv
        dzbuf[r + 8, pl.ds(0, 16)] = zv
        dzbuf2[r, pl.ds(0, 16)] = zv
    r0 = s * ROWS_PER_TILE
    d0 = s * DROWS_PER_TILE
    if _STAGE >= 1:
        for t in range(ROWS_PER_TILE // 8):
            pltpu.sync_copy(zbuf, acc_sh.at[pl.ds(r0 + t * 8, 8)])
        for t in range(DROWS_PER_TILE // 16):
            pltpu.sync_copy(dzbuf, den_sh.at[pl.ds(d0 + t * 16, 16)])
        pltpu.sync_copy(dzbuf2,
                        den_sh.at[pl.ds(d0 + (DROWS_PER_TILE // 16) * 16, 8)])
    if _STAGE >= 2:
        plsc.subcore_barrier()

    def _block(sb, carry0):
        row = s * SB + sb
        pltpu.sync_copy(srcm.at[row], src_st)
        pltpu.sync_copy(dstm.at[row], dst_st)

        def _window(w, carry):
            base = pl.multiple_of(w * W, W)
            for i in range(W // 16):
                dv = dst_st[pl.ds(base + i * 16, 16)]
                sv = src_st[pl.ds(base + i * 16, 16)]
                m = (dv >= lo) & (dv < lo + HALF)
                dloc = jnp.where(m, dv - lo, jnp.int32(HALF))
                sidx[pl.ds(i * 16, 16)] = sv
                qiw[pl.ds(i * 16, 16)] = jnp.minimum(dv, jnp.int32(N_NODE - 1))
                dstw[pl.ds(i * 16, 16)] = dloc
                dstw2[pl.ds(i * 16, 16)] = dloc >> 2
            if _STAGE >= 5:
                cpk = pltpu.async_copy(kp.at[sidx], kbuf, sem_k)
                cpq = pltpu.async_copy(qm.at[qiw], qbuf, sem_q)
                cpv = pltpu.async_copy(vp.at[sidx], upd, sem_v)
                cpk.wait()
                cpq.wait()
                cpv.wait()

            if _STAGE >= 6:
                def _group(g, carry2):
                    goff = pl.multiple_of(g * 16, 16)
                    dvec = dstw[pl.ds(goff, 16)]
                    for j in range(16):
                        e = goff + j
                        ex = []
                        for h in range(H):
                            k0 = kbuf[e, pl.ds(h * 32, 16)]
                            k1 = kbuf[e, pl.ds(h * 32 + 16, 16)]
                            q0 = qbuf[e, pl.ds(h * 32, 16)]
                            q1 = qbuf[e, pl.ds(h * 32 + 16, 16)]
                            th = k0 * q0 + k1 * q1
                            ex.append(jnp.exp(_hsum_splat(th, lane16)))
                        for i in range(D // 16):
                            upd[e, pl.ds(i * 16, 16)] = upd[e, pl.ds(i * 16, 16)] * ex[i // 2]
                        exr = jnp.where(lmod4 == 0, ex[0],
                                        jnp.where(lmod4 == 1, ex[1],
                                                  jnp.where(lmod4 == 2, ex[2], ex[3])))
                        dsplat = _perm(dvec, jnp.full((16,), j, jnp.int32))
                        dmod = dsplat & 3
                        dupd[e, pl.ds(0, 16)] = jnp.where(lgrp4 == dmod, exr, 0.0)
                    return carry2

                lax.fori_loop(0, W // 16, _group, 0)
            if _STAGE >= 7:
                pltpu.sync_copy(upd, acc_sh.at[dstw], add=True)
            if _STAGE >= 8:
                pltpu.sync_copy(dupd, den_sh.at[dstw2], add=True)
            return carry

        if _STAGE >= 5:
            lax.fori_loop(0, WPB, _window, 0)
        return carry0

    if _STAGE >= 4:
        lax.fori_loop(0, SB, _block, 0)

    if _STAGE >= 2:
        plsc.subcore_barrier()
    if _STAGE >= 3:
        for cc in range(NC):
            @pl.when(c == cc)
            def _():
                pltpu.sync_copy(acc_sh.at[pl.ds(r0, ROWS_PER_TILE)],
                                out_agg.at[cc, pl.ds(r0, ROWS_PER_TILE)])
                pltpu.sync_copy(den_sh.at[pl.ds(d0, DROWS_PER_TILE)],
                                out_den.at[cc, pl.ds(d0, DROWS_PER_TILE)])


@functools.partial(
    pl.kernel,
    out_type=jax.ShapeDtypeStruct((B, D), jnp.float32),
    mesh=_SC_MESH,
    scratch_types=[
        pltpu.VMEM((B // (NC * NS),), jnp.int32),
        pltpu.VMEM((128,), jnp.int32),
        pltpu.VMEM((128, D), jnp.float32),
        pltpu.SemaphoreType.DMA,
    ],
)
def _gather_kernel(table, idx, out, ibuf, iwin, rbuf, sem):
    wid = lax.axis_index("s") * NC + lax.axis_index("c")
    per = B // (NC * NS)
    base = wid * per
    pltpu.sync_copy(idx.at[pl.ds(base, per)], ibuf)
    for w in range(per // 128):
        for i in range(8):
            iwin[pl.ds(i * 16, 16)] = ibuf[pl.ds(w * 128 + i * 16, 16)]
        pltpu.async_copy(table.at[iwin], rbuf, sem).wait()
        pltpu.sync_copy(rbuf, out.at[pl.ds(base + w * 128, 128)])


def _block_diag(rel):
    eye = jnp.eye(H, dtype=jnp.float32)
    t = eye[None, None, :, None, :, None] * rel[:, :, :, :, None, :]
    return t.reshape(L, 2, D, D)


def kernel(x_drug, x_disease, W_in, b_in, Wk, bk, Wq, bq, Wv, bv, Wo, bo,
           a_rel, m_rel, p_rel, skip, lng, lnb, W1, b1, W2, b2, W3, b3,
           ei_treats, ei_rev, drug_index, disease_index):
    f32 = jnp.float32
    Abd = _block_diag(a_rel)
    Mbd = _block_diag(m_rel)
    Wk_f = jnp.einsum('ltij,ltjk->ltik', Wk, Abd)
    bk_f = jnp.einsum('ltj,ltjk->ltk', bk, Abd)
    Wv_f = jnp.einsum('ltij,ltjk->ltik', Wv, Mbd)
    bv_f = jnp.einsum('ltj,ltjk->ltk', bv, Mbd)
    qs = jnp.repeat(p_rel[:, ::-1, :], DH, axis=-1) * (1.0 / jnp.sqrt(DH))
    Wq_f = Wq * qs[:, :, None, :]
    bq_f = bq * qs
    beta = jax.nn.sigmoid(skip)


    edge_idx = [ei_treats, ei_rev]
    xs = [None, None]
    K = [None, None]
    Q = [None, None]
    V = [None, None]
    x_in = [x_drug, x_disease]
    for l in range(L):
        for t in range(2):
            if l == 0:
                xs[t], K[t], Q[t], V[t] = _proj_kqv(
                    x_in[t], W_in[t], b_in[t],
                    Wk_f[l, t], bk_f[l, t], Wq_f[l, t], bq_f[l, t],
                    Wv_f[l, t], bv_f[l, t])
            else:
                K[t], Q[t], V[t] = _kqv(
                    xs[t], Wk_f[l, t], bk_f[l, t], Wq_f[l, t], bq_f[l, t],
                    Wv_f[l, t], bv_f[l, t])
        acc = [None, None]
        den = [None, None]
        for e in range(2):
            src, dst = edge_idx[e][0], edge_idx[e][1]
            k_e = K[e].reshape(N_NODE, H, DH)[src]
            v_e = V[e].reshape(N_NODE, H, DH)[src]
            q_e = Q[1 - e].reshape(N_NODE, H, DH)[dst]
            exv = jnp.exp(jnp.sum(q_e * k_e, axis=-1))
            den[e] = jax.ops.segment_sum(exv, dst, num_segments=N_NODE)
            acc[e] = jax.ops.segment_sum(
                v_e * exv[:, :, None], dst, num_segments=N_NODE
            ).reshape(N_NODE, D)
        new_xs = []
        for t in range(2):
            beta_t = beta[l, t].astype(f32).reshape(1, 1)
            new_xs.append(_post(acc[1 - t], den[1 - t], xs[t], Wo[l, t],
                                bo[l, t], beta_t, lng[l, t], lnb[l, t]))
        xs = new_xs

    g0 = _gather_kernel(xs[0], drug_index)
    g1 = _gather_kernel(xs[1], disease_index)
    return _decoder_mlp(g0, g1, W1, b1, W2, b2, W3, b3)
